# baseline (speedup 1.0000x reference)
import functools

import jax
import jax.numpy as jnp
from jax import lax
from jax.experimental import pallas as pl
from jax.experimental.pallas import tpu as pltpu
from jax.experimental.pallas import tpu_sc as plsc

_NC = 2
_NS = 16
_NW = _NC * _NS
_LANES = 16
_CH = 128


def _in_body(x_ref, o_ref):
    h2 = x_ref.shape[1] // 2
    y = jnp.swapaxes(x_ref[...], 0, 1)
    o_ref[...] = jnp.concatenate([y[0:h2], y[h2:2 * h2]], axis=1)


def _out_body(x_ref, o_ref):
    d = o_ref.shape[0]
    h2 = o_ref.shape[1] // 2
    x = x_ref[...]
    y = jnp.concatenate([x[:, 0:d], x[:, d:2 * d]], axis=0)
    o_ref[...] = jnp.swapaxes(y, 0, 1)


_CB = 8192


def _mphys(m):
    return ((m + _CB - 1) // _CB) * _CB


@functools.lru_cache(maxsize=None)
def _make_in_convert(m, d):
    ob = _CB * d // 128
    grid = (m + _CB - 1) // _CB
    return pl.pallas_call(
        _in_body,
        grid=(grid,),
        in_specs=[pl.BlockSpec((d, _CB), lambda i: (0, i))],
        out_specs=pl.BlockSpec((ob, 128), lambda i: (i, 0)),
        out_shape=jax.ShapeDtypeStruct((_mphys(m) * d // 128, 128),
                                       jnp.float32),
    )


@functools.lru_cache(maxsize=None)
def _make_out_convert(m, d):
    ob = _CB * d // 128
    grid = (m + _CB - 1) // _CB
    return pl.pallas_call(
        _out_body,
        grid=(grid,),
        in_specs=[pl.BlockSpec((ob, 128), lambda i: (i, 0))],
        out_specs=pl.BlockSpec((d, _CB), lambda i: (0, i)),
        out_shape=jax.ShapeDtypeStruct((d, m), jnp.float32),
    )


def _mesh():
    return plsc.VectorSubcoreMesh(
        core_axis_name="c", subcore_axis_name="s",
        num_cores=_NC, num_subcores=_NS,
    )


@functools.lru_cache(maxsize=None)
def _make_sc_scan(m, d, b):
    assert m % _NW == 0
    rpw = m // _NW
    lpad = ((rpw + 15) // 16) * 16
    ngrp = b // _LANES
    nch_max = (b + _CH - 1) // _CH

    def body(idx_hbm, val_hbm, obs_hbm, wl_hbm, rl_hbm, cnt_hbm,
             idx_v, L_v, wlist, rlist, jlist, buf, cnt_v,
             sem_g, sem_s2):
        wid = lax.axis_index("s") * _NC + lax.axis_index("c")
        base = wid * rpw
        lane = lax.iota(jnp.int32, _LANES)

        pltpu.sync_copy(idx_hbm, idx_v)

        neg1 = jnp.full((_LANES,), -1, jnp.int32)

        def init_body(i, carry):
            L_v[pl.ds(i * _LANES, _LANES)] = neg1
            return carry

        lax.fori_loop(0, lpad // _LANES, init_body, 0)

        def scan_body(g, carry):
            jv = lane + g * _LANES
            iv = idx_v[pl.ds(g * _LANES, _LANES)]
            inr = (iv >= base) & (iv < base + rpw)
            r = jnp.where(inr, iv - base, 0)
            plsc.store_scatter(L_v, [r], jv, mask=inr)
            got = plsc.load_gather(L_v, [r], mask=inr)
            m0 = jnp.where(inr & (got < jv), 1, 0).astype(jnp.int32)

            def fix_cond(mm):
                return jnp.max(mm, axis=0) > 0

            def fix_body(mm):
                plsc.store_scatter(L_v, [r], jv, mask=mm > 0)
                got2 = plsc.load_gather(L_v, [r], mask=inr)
                return jnp.where(inr & (got2 < jv), 1, 0).astype(jnp.int32)

            lax.while_loop(fix_cond, fix_body, m0)
            return carry

        lax.fori_loop(0, ngrp, scan_body, 0)

        def comp_body(g, cnt):
            jv = lane + g * _LANES
            iv = idx_v[pl.ds(g * _LANES, _LANES)]
            inr = (iv >= base) & (iv < base + rpw)
            r = jnp.where(inr, iv - base, 0)
            w = plsc.load_gather(L_v, [r], mask=inr)
            inc = jnp.where(inr, 1, 0).astype(jnp.int32)
            pos = cnt + plsc.cumsum(inc) - 1
            pos = jnp.where(inr, pos, 0)
            row = lax.shift_right_logical(pos, 7)
            col = pos & (_CH - 1)
            pv = ((iv & ~(_CB - 1)) | ((iv & (_CB // 2 - 1)) << 1)
                  | (lax.shift_right_logical(iv, _CB.bit_length() - 2) & 1))
            plsc.store_scatter(wlist, [row, col], w, mask=inr)
            plsc.store_scatter(rlist, [row, col], pv, mask=inr)
            plsc.store_scatter(jlist, [row, col], jv, mask=inr)
            npop = plsc.all_reduce_population_count(inr)
            return cnt + npop[0]

        cnt = lax.fori_loop(0, ngrp, comp_body, jnp.int32(0))

        @pl.when(cnt > 0)
        def _():
            lastrow = lax.shift_right_logical(cnt - 1, 7)
            lastcol = (cnt - 1) & (_CH - 1)
            rowv = jnp.zeros((_LANES,), jnp.int32) + lastrow
            colv = jnp.zeros((_LANES,), jnp.int32) + lastcol
            wpad = plsc.load_gather(wlist, [rowv, colv])
            rpad = plsc.load_gather(rlist, [rowv, colv])
            jpad = plsc.load_gather(jlist, [rowv, colv])
            for t in range(_CH // _LANES):
                cols = lane + t * _LANES
                mpad = (lastrow * _CH + cols) >= cnt
                plsc.store_scatter(wlist, [rowv, cols], wpad, mask=mpad)
                plsc.store_scatter(rlist, [rowv, cols], rpad, mask=mpad)
                plsc.store_scatter(jlist, [rowv, cols], jpad, mask=mpad)

            nch = lax.shift_right_logical(cnt + _CH - 1, 7)

            def dma_body(cidx, carry):
                pltpu.async_copy(val_hbm.at[wlist.at[cidx]], buf, sem_g).wait()
                pltpu.async_copy(buf, obs_hbm.at[jlist.at[cidx]],
                                 sem_s2).wait()
                return carry

            lax.fori_loop(0, nch, dma_body, 0)

        cnt_v[...] = jnp.zeros((_LANES,), jnp.int32) + cnt
        pltpu.sync_copy(wlist, wl_hbm.at[wid])
        pltpu.sync_copy(rlist, rl_hbm.at[wid])
        pltpu.sync_copy(cnt_v, cnt_hbm.at[wid])

    return pl.kernel(
        body,
        out_type=(
            jax.ShapeDtypeStruct((b, d), jnp.float32),
            jax.ShapeDtypeStruct((_NW, nch_max, _CH), jnp.int32),
            jax.ShapeDtypeStruct((_NW, nch_max, _CH), jnp.int32),
            jax.ShapeDtypeStruct((_NW, _LANES), jnp.int32),
        ),
        mesh=_mesh(),
        scratch_types=[
            pltpu.VMEM((b,), jnp.int32),
            pltpu.VMEM((lpad,), jnp.int32),
            pltpu.VMEM((nch_max, _CH), jnp.int32),
            pltpu.VMEM((nch_max, _CH), jnp.int32),
            pltpu.VMEM((nch_max, _CH), jnp.int32),
            pltpu.VMEM((_CH, d), jnp.float32),
            pltpu.VMEM((_LANES,), jnp.int32),
            pltpu.SemaphoreType.DMA,
            pltpu.SemaphoreType.DMA,
        ],
        compiler_params=pltpu.CompilerParams(
            needs_layout_passes=False, use_tc_tiling_on_sc=False),
    )


@functools.lru_cache(maxsize=None)
def _make_sc_apply(m, d, b):
    nch_max = (b + _CH - 1) // _CH

    def body(newmem_ref, val_hbm, wl_hbm, rl_hbm, cnt_hbm,
             wlist, rlist, cnt_v, buf, sem_g, sem_s1):
        wid = lax.axis_index("s") * _NC + lax.axis_index("c")
        pltpu.sync_copy(wl_hbm.at[wid], wlist)
        pltpu.sync_copy(rl_hbm.at[wid], rlist)
        pltpu.sync_copy(cnt_hbm.at[wid], cnt_v)
        cnt = cnt_v[...][0]

        @pl.when(cnt > 0)
        def _():
            nch = lax.shift_right_logical(cnt + _CH - 1, 7)

            def dma_body(cidx, carry):
                pltpu.async_copy(val_hbm.at[wlist.at[cidx]], buf, sem_g).wait()
                pltpu.async_copy(buf, newmem_ref.at[rlist.at[cidx]],
                                 sem_s1).wait()
                return carry

            lax.fori_loop(0, nch, dma_body, 0)

    return pl.kernel(
        body,
        out_type=(),
        mesh=_mesh(),
        scratch_types=[
            pltpu.VMEM((nch_max, _CH), jnp.int32),
            pltpu.VMEM((nch_max, _CH), jnp.int32),
            pltpu.VMEM((_LANES,), jnp.int32),
            pltpu.VMEM((_CH, d), jnp.float32),
            pltpu.SemaphoreType.DMA,
            pltpu.SemaphoreType.DMA,
        ],
        compiler_params=pltpu.CompilerParams(
            needs_layout_passes=False, use_tc_tiling_on_sc=False),
    )


def kernel(mem, idx, val):
    m, d = mem.shape
    b = idx.shape[0]
    mp = _mphys(m)
    flat = _make_in_convert(m, d)(mem.T)
    observed, wl, rl, cnts = _make_sc_scan(m, d, b)(idx, val)
    mem_ref = jax.new_ref(flat.reshape(mp, d))
    _make_sc_apply(m, d, b)(mem_ref, val, wl, rl, cnts)
    new_memT = _make_out_convert(m, d)(
        mem_ref[...].reshape(mp * d // 128, 128))
    return new_memT.T, observed


# speedup vs baseline: 1.1986x; 1.1986x over previous
import functools

import jax
import jax.numpy as jnp
from jax import lax
from jax.experimental import pallas as pl
from jax.experimental.pallas import tpu as pltpu
from jax.experimental.pallas import tpu_sc as plsc

_NC = 2
_NS = 16
_NW = _NC * _NS
_LANES = 16
_CH = 128


def _in_body(x_ref, o_ref):
    h2 = x_ref.shape[1] // 2
    y = jnp.swapaxes(x_ref[...], 0, 1)
    o_ref[...] = jnp.concatenate([y[0:h2], y[h2:2 * h2]], axis=1)


def _out_body(x_ref, o_ref):
    d = o_ref.shape[0]
    h2 = o_ref.shape[1] // 2
    x = x_ref[...]
    y = jnp.concatenate([x[:, 0:d], x[:, d:2 * d]], axis=0)
    o_ref[...] = jnp.swapaxes(y, 0, 1)


_CB = 32768


def _mphys(m):
    return ((m + _CB - 1) // _CB) * _CB


@functools.lru_cache(maxsize=None)
def _make_in_convert(m, d):
    ob = _CB * d // 128
    grid = (m + _CB - 1) // _CB
    return pl.pallas_call(
        _in_body,
        grid=(grid,),
        in_specs=[pl.BlockSpec((d, _CB), lambda i: (0, i))],
        out_specs=pl.BlockSpec((ob, 128), lambda i: (i, 0)),
        out_shape=jax.ShapeDtypeStruct((_mphys(m) * d // 128, 128),
                                       jnp.float32),
    )


@functools.lru_cache(maxsize=None)
def _make_out_convert(m, d):
    ob = _CB * d // 128
    grid = (m + _CB - 1) // _CB
    return pl.pallas_call(
        _out_body,
        grid=(grid,),
        in_specs=[pl.BlockSpec((ob, 128), lambda i: (i, 0))],
        out_specs=pl.BlockSpec((d, _CB), lambda i: (0, i)),
        out_shape=jax.ShapeDtypeStruct((d, m), jnp.float32),
    )


def _mesh():
    return plsc.VectorSubcoreMesh(
        core_axis_name="c", subcore_axis_name="s",
        num_cores=_NC, num_subcores=_NS,
    )


@functools.lru_cache(maxsize=None)
def _make_sc_scan(m, d, b):
    assert m % _NW == 0
    rpw = m // _NW
    lpad = ((rpw + 15) // 16) * 16
    ngrp = b // _LANES
    nch_max = (b + _CH - 1) // _CH

    def body(idx_hbm, val_hbm, obs_hbm, wl_hbm, rl_hbm, cnt_hbm,
             idx_v, L_v, wlist, rlist, jlist, buf, cnt_v,
             sem_g, sem_s2):
        wid = lax.axis_index("s") * _NC + lax.axis_index("c")
        base = wid * rpw
        lane = lax.iota(jnp.int32, _LANES)

        pltpu.sync_copy(idx_hbm, idx_v)

        neg1 = jnp.full((_LANES,), -1, jnp.int32)

        def init_body(i, carry):
            L_v[pl.ds(i * _LANES, _LANES)] = neg1
            return carry

        lax.fori_loop(0, lpad // _LANES, init_body, 0)

        def scan_body(g, carry):
            jv = lane + g * _LANES
            iv = idx_v[pl.ds(g * _LANES, _LANES)]
            inr = (iv >= base) & (iv < base + rpw)
            r = jnp.where(inr, iv - base, 0)
            plsc.store_scatter(L_v, [r], jv, mask=inr)
            got = plsc.load_gather(L_v, [r], mask=inr)
            m0 = jnp.where(inr & (got < jv), 1, 0).astype(jnp.int32)

            def fix_cond(mm):
                return jnp.max(mm, axis=0) > 0

            def fix_body(mm):
                plsc.store_scatter(L_v, [r], jv, mask=mm > 0)
                got2 = plsc.load_gather(L_v, [r], mask=inr)
                return jnp.where(inr & (got2 < jv), 1, 0).astype(jnp.int32)

            lax.while_loop(fix_cond, fix_body, m0)
            return carry

        lax.fori_loop(0, ngrp, scan_body, 0)

        def comp_body(g, cnt):
            jv = lane + g * _LANES
            iv = idx_v[pl.ds(g * _LANES, _LANES)]
            inr = (iv >= base) & (iv < base + rpw)
            r = jnp.where(inr, iv - base, 0)
            w = plsc.load_gather(L_v, [r], mask=inr)
            inc = jnp.where(inr, 1, 0).astype(jnp.int32)
            pos = cnt + plsc.cumsum(inc) - 1
            pos = jnp.where(inr, pos, 0)
            row = lax.shift_right_logical(pos, 7)
            col = pos & (_CH - 1)
            pv = ((iv & ~(_CB - 1)) | ((iv & (_CB // 2 - 1)) << 1)
                  | (lax.shift_right_logical(iv, _CB.bit_length() - 2) & 1))
            plsc.store_scatter(wlist, [row, col], w, mask=inr)
            plsc.store_scatter(rlist, [row, col], pv, mask=inr)
            plsc.store_scatter(jlist, [row, col], jv, mask=inr)
            npop = plsc.all_reduce_population_count(inr)
            return cnt + npop[0]

        cnt = lax.fori_loop(0, ngrp, comp_body, jnp.int32(0))

        @pl.when(cnt > 0)
        def _():
            lastrow = lax.shift_right_logical(cnt - 1, 7)
            lastcol = (cnt - 1) & (_CH - 1)
            rowv = jnp.zeros((_LANES,), jnp.int32) + lastrow
            colv = jnp.zeros((_LANES,), jnp.int32) + lastcol
            wpad = plsc.load_gather(wlist, [rowv, colv])
            rpad = plsc.load_gather(rlist, [rowv, colv])
            jpad = plsc.load_gather(jlist, [rowv, colv])
            for t in range(_CH // _LANES):
                cols = lane + t * _LANES
                mpad = (lastrow * _CH + cols) >= cnt
                plsc.store_scatter(wlist, [rowv, cols], wpad, mask=mpad)
                plsc.store_scatter(rlist, [rowv, cols], rpad, mask=mpad)
                plsc.store_scatter(jlist, [rowv, cols], jpad, mask=mpad)

            nch = lax.shift_right_logical(cnt + _CH - 1, 7)

            def dma_body(cidx, carry):
                pltpu.async_copy(val_hbm.at[wlist.at[cidx]], buf, sem_g).wait()
                pltpu.async_copy(buf, obs_hbm.at[jlist.at[cidx]],
                                 sem_s2).wait()
                return carry

            lax.fori_loop(0, nch, dma_body, 0)

        cnt_v[...] = jnp.zeros((_LANES,), jnp.int32) + cnt
        pltpu.sync_copy(wlist, wl_hbm.at[wid])
        pltpu.sync_copy(rlist, rl_hbm.at[wid])
        pltpu.sync_copy(cnt_v, cnt_hbm.at[wid])

    return pl.kernel(
        body,
        out_type=(
            jax.ShapeDtypeStruct((b, d), jnp.float32),
            jax.ShapeDtypeStruct((_NW, nch_max, _CH), jnp.int32),
            jax.ShapeDtypeStruct((_NW, nch_max, _CH), jnp.int32),
            jax.ShapeDtypeStruct((_NW, _LANES), jnp.int32),
        ),
        mesh=_mesh(),
        scratch_types=[
            pltpu.VMEM((b,), jnp.int32),
            pltpu.VMEM((lpad,), jnp.int32),
            pltpu.VMEM((nch_max, _CH), jnp.int32),
            pltpu.VMEM((nch_max, _CH), jnp.int32),
            pltpu.VMEM((nch_max, _CH), jnp.int32),
            pltpu.VMEM((_CH, d), jnp.float32),
            pltpu.VMEM((_LANES,), jnp.int32),
            pltpu.SemaphoreType.DMA,
            pltpu.SemaphoreType.DMA,
        ],
        compiler_params=pltpu.CompilerParams(
            needs_layout_passes=False, use_tc_tiling_on_sc=False),
    )


@functools.lru_cache(maxsize=None)
def _make_sc_apply(m, d, b):
    nch_max = (b + _CH - 1) // _CH

    def body(newmem_ref, val_hbm, wl_hbm, rl_hbm, cnt_hbm,
             wlist, rlist, cnt_v, buf_a, buf_b, sem_ga, sem_gb, sem_s1):
        wid = lax.axis_index("s") * _NC + lax.axis_index("c")
        pltpu.sync_copy(wl_hbm.at[wid], wlist)
        pltpu.sync_copy(rl_hbm.at[wid], rlist)
        pltpu.sync_copy(cnt_hbm.at[wid], cnt_v)
        cnt = cnt_v[...][0]

        @pl.when(cnt > 0)
        def _():
            nch = lax.shift_right_logical(cnt + _CH - 1, 7)
            pltpu.async_copy(val_hbm.at[wlist.at[0]], buf_a, sem_ga)

            def step(cur, nxt, cidx, sem_cur, sem_nxt):
                pltpu.make_async_copy(val_hbm.at[wlist.at[cidx]], cur,
                                      sem_cur).wait()

                @pl.when(cidx + 1 < nch)
                def _():
                    pltpu.async_copy(val_hbm.at[wlist.at[cidx + 1]], nxt,
                                     sem_nxt)

                pltpu.async_copy(cur, newmem_ref.at[rlist.at[cidx]],
                                 sem_s1).wait()

            def dma_body(cidx, carry):
                @pl.when(cidx % 2 == 0)
                def _():
                    step(buf_a, buf_b, cidx, sem_ga, sem_gb)

                @pl.when(cidx % 2 == 1)
                def _():
                    step(buf_b, buf_a, cidx, sem_gb, sem_ga)

                return carry

            lax.fori_loop(0, nch, dma_body, 0)

    return pl.kernel(
        body,
        out_type=(),
        mesh=_mesh(),
        scratch_types=[
            pltpu.VMEM((nch_max, _CH), jnp.int32),
            pltpu.VMEM((nch_max, _CH), jnp.int32),
            pltpu.VMEM((_LANES,), jnp.int32),
            pltpu.VMEM((_CH, d), jnp.float32),
            pltpu.VMEM((_CH, d), jnp.float32),
            pltpu.SemaphoreType.DMA,
            pltpu.SemaphoreType.DMA,
            pltpu.SemaphoreType.DMA,
        ],
        compiler_params=pltpu.CompilerParams(
            needs_layout_passes=False, use_tc_tiling_on_sc=False),
    )


def kernel(mem, idx, val):
    m, d = mem.shape
    b = idx.shape[0]
    mp = _mphys(m)
    flat = _make_in_convert(m, d)(mem.T)
    observed, wl, rl, cnts = _make_sc_scan(m, d, b)(idx, val)
    mem_ref = jax.new_ref(flat.reshape(mp, d))
    _make_sc_apply(m, d, b)(mem_ref, val, wl, rl, cnts)
    new_memT = _make_out_convert(m, d)(
        mem_ref[...].reshape(mp * d // 128, 128))
    return new_memT.T, observed
